# baseline (speedup 1.0000x reference)
import functools

import jax
import jax.numpy as jnp
from jax import lax
from jax.experimental import pallas as pl
from jax.experimental.pallas import tpu as pltpu
from jax.experimental.pallas import tpu_sc as plsc

_VOCAB = 1000
_D = 128
_B = 1024
_L = 50
_CB = 64
_NBC = _B // _CB
_NW = 32
_L0 = 24
_L1 = _L - _L0


def _make_gather(nch):
    mesh = plsc.VectorSubcoreMesh(core_axis_name="c", subcore_axis_name="s")

    @functools.partial(
        pl.kernel,
        out_type=jax.ShapeDtypeStruct((_NW * nch, _CB, _D), jnp.float32),
        mesh=mesh,
        scratch_types=[
            pltpu.VMEM((nch, _CB), jnp.int32),
            pltpu.VMEM((_CB, _D), jnp.float32),
            pltpu.VMEM((_CB, _D), jnp.float32),
            pltpu.SemaphoreType.DMA,
            pltpu.SemaphoreType.DMA,
        ],
    )
    def gather(ids_hbm, emb_hbm, x_hbm, idx_v, buf0, buf1, sem0, sem1):
        wid = lax.axis_index("s") * 2 + lax.axis_index("c")
        base = wid * nch
        pltpu.sync_copy(ids_hbm.at[wid], idx_v)

        def g(c, buf, sem):
            return pltpu.make_async_copy(emb_hbm.at[idx_v.at[c]], buf, sem)

        def w(c, buf):
            pltpu.sync_copy(buf, x_hbm.at[base + c])

        g(0, buf0, sem0).start()

        def body(p, carry):
            a = 2 * p
            g(a + 1, buf1, sem1).start()
            g(a, buf0, sem0).wait()
            w(a, buf0)
            g(a + 2, buf0, sem0).start()
            g(a + 1, buf1, sem1).wait()
            w(a + 1, buf1)
            return carry

        if nch % 2 == 0:
            lax.fori_loop(0, nch // 2 - 1, body, 0)
            g(nch - 1, buf1, sem1).start()
            g(nch - 2, buf0, sem0).wait()
            w(nch - 2, buf0)
            g(nch - 1, buf1, sem1).wait()
            w(nch - 1, buf1)
        else:
            lax.fori_loop(0, (nch - 3) // 2, body, 0)
            g(nch - 2, buf1, sem1).start()
            g(nch - 3, buf0, sem0).wait()
            w(nch - 3, buf0)
            g(nch - 1, buf0, sem0).start()
            g(nch - 2, buf1, sem1).wait()
            w(nch - 2, buf1)
            g(nch - 1, buf0, sem0).wait()
            w(nch - 1, buf0)

    return gather


_gather0 = _make_gather(_L0 * _NBC // _NW)
_gather1 = _make_gather(_L1 * _NBC // _NW)


def _head_body(x_ref, w_ref, b_ref, out_ref):
    xl = x_ref[...].reshape(_B, _D)
    res = lax.dot_general(
        w_ref[...], xl, (((1,), (1,)), ((), ())),
        preferred_element_type=jnp.float32)
    out_ref[...] = (res + b_ref[...]).reshape(1, _VOCAB, _B)


def _head0(x, w, b2d):
    return pl.pallas_call(
        _head_body,
        grid=(_L0,),
        in_specs=[
            pl.BlockSpec((_NBC, _CB, _D), lambda i: (i, 0, 0)),
            pl.BlockSpec((_VOCAB, _D), lambda i: (0, 0)),
            pl.BlockSpec((_VOCAB, 1), lambda i: (0, 0)),
        ],
        out_specs=pl.BlockSpec((1, _VOCAB, _B), lambda i: (i, 0, 0)),
        out_shape=jax.ShapeDtypeStruct((_L, _VOCAB, _B), jnp.float32),
    )(x, w, b2d)


def _head1_body(x_ref, w_ref, b_ref, prev_ref, out_ref):
    del prev_ref
    _head_body(x_ref, w_ref, b_ref, out_ref)


def _head1(x, w, b2d, prev):
    return pl.pallas_call(
        _head1_body,
        grid=(_L1,),
        in_specs=[
            pl.BlockSpec((_NBC, _CB, _D), lambda i: (i, 0, 0)),
            pl.BlockSpec((_VOCAB, _D), lambda i: (0, 0)),
            pl.BlockSpec((_VOCAB, 1), lambda i: (0, 0)),
            pl.BlockSpec((1, _VOCAB, _B), lambda i: (i + _L0, 0, 0)),
        ],
        out_specs=pl.BlockSpec((1, _VOCAB, _B), lambda i: (i + _L0, 0, 0)),
        out_shape=jax.ShapeDtypeStruct((_L, _VOCAB, _B), jnp.float32),
        input_output_aliases={3: 0},
    )(x, w, b2d, prev)


def kernel(input_ids, emb_table, head_w, head_b):
    ids = input_ids.astype(jnp.int32)
    ids_t = ids.T.reshape(_L * _NBC, _CB)
    n0 = _L0 * _NBC
    ids0 = ids_t[:n0].reshape(_NW, -1, _CB)
    ids1 = ids_t[n0:].reshape(_NW, -1, _CB)
    b2d = head_b.reshape(_VOCAB, 1)
    x0 = _gather0(ids0, emb_table)
    x1 = _gather1(ids1, emb_table)
    out_t = _head0(x0, head_w, b2d)
    out_t = _head1(x1, head_w, b2d, out_t)
    return jnp.transpose(out_t, (2, 0, 1))
